# Initial kernel scaffold; baseline (speedup 1.0000x reference)
import jax
import jax.numpy as jnp
from jax.experimental import pallas as pl


def kernel(x, edge_index, edge_weight, W1l, b1l, W1r, W2l, b2l, W2r):
    raise NotImplementedError("write your pallas kernel here")


# speedup vs baseline: 12.1784x; 12.1784x over previous
import functools

import jax
import jax.numpy as jnp
from jax import lax
from jax.experimental import pallas as pl
from jax.experimental.pallas import tpu as pltpu
from jax.experimental.pallas import tpu_sc as plsc

NC = 2
NS = 16
NW = NC * NS
CH = 128


def _make_agg(n_nodes, e2d_rows, k, with_count):
    wrows = e2d_rows // NW
    iters = wrows // k
    assert wrows % k == 0
    nacc = -(-(n_nodes + 1) // (NS * 8)) * NS * 8
    zsl = nacc // NS
    f32 = jnp.float32

    outs = [jax.ShapeDtypeStruct((NC, nacc, 16), f32)]
    scratch = [
        pltpu.VMEM((k, CH), jnp.int32),
        pltpu.VMEM((k, CH), jnp.int32),
        pltpu.VMEM((k * CH, 16), f32),
        pltpu.VMEM_SHARED((nacc, 16), f32),
        pltpu.SemaphoreType.DMA,
    ]
    if with_count:
        outs = outs + [jax.ShapeDtypeStruct((NC, nacc, 16), f32)]
        scratch = scratch + [
            pltpu.VMEM((CH, 16), f32),
            pltpu.VMEM_SHARED((nacc, 16), f32),
        ]

    def body(*refs):
        if with_count:
            (table, src2d, dst2d, zeros_h, ones_h, out_acc, out_cnt,
             srcb, dstb, rowsb, acc, sem, onesb, cacc) = refs
        else:
            (table, src2d, dst2d, zeros_h, out_acc,
             srcb, dstb, rowsb, acc, sem) = refs
        c = lax.axis_index("c")
        s = lax.axis_index("s")
        w = s * NC + c

        pltpu.sync_copy(zeros_h.at[pl.ds(s * zsl, zsl)], acc.at[pl.ds(s * zsl, zsl)])
        if with_count:
            pltpu.sync_copy(zeros_h.at[pl.ds(s * zsl, zsl)], cacc.at[pl.ds(s * zsl, zsl)])
            pltpu.sync_copy(ones_h, onesb)
        plsc.subcore_barrier()

        base = w * wrows

        def it_body(it, carry):
            r0 = base + it * k
            pltpu.sync_copy(src2d.at[pl.ds(r0, k)], srcb)
            pltpu.sync_copy(dst2d.at[pl.ds(r0, k)], dstb)
            descs = [
                pltpu.async_copy(table.at[srcb.at[j]],
                                 rowsb.at[pl.ds(j * CH, CH)], sem)
                for j in range(k)
            ]
            for d in descs:
                d.wait()
            for j in range(k):
                pltpu.sync_copy(rowsb.at[pl.ds(j * CH, CH)],
                                acc.at[dstb.at[j]], add=True)
            if with_count:
                for j in range(k):
                    pltpu.sync_copy(onesb, cacc.at[dstb.at[j]], add=True)
            return carry

        lax.fori_loop(0, iters, it_body, 0)
        plsc.subcore_barrier()

        pltpu.sync_copy(acc.at[pl.ds(s * zsl, zsl)],
                        out_acc.at[c, pl.ds(s * zsl, zsl)])
        if with_count:
            pltpu.sync_copy(cacc.at[pl.ds(s * zsl, zsl)],
                            out_cnt.at[c, pl.ds(s * zsl, zsl)])

    return pl.kernel(
        body,
        out_type=tuple(outs),
        mesh=plsc.VectorSubcoreMesh(core_axis_name="c", subcore_axis_name="s"),
        scratch_types=tuple(scratch),
        compiler_params=pltpu.CompilerParams(use_tc_tiling_on_sc=False),
    )


def _p1(x, wcat_t, bm=2000):
    n, f_in = x.shape

    def body(x_ref, w_ref, z_ref, r_ref):
        res = jnp.dot(x_ref[...], w_ref[...], preferred_element_type=jnp.float32)
        z_ref[...] = res[:, :16]
        r_ref[...] = res[:, 16:]

    return pl.pallas_call(
        body,
        grid=(n // bm,),
        in_specs=[pl.BlockSpec((bm, f_in), lambda i: (i, 0)),
                  pl.BlockSpec((f_in, 32), lambda i: (0, 0))],
        out_specs=[pl.BlockSpec((bm, 16), lambda i: (i, 0)),
                   pl.BlockSpec((bm, 16), lambda i: (i, 0))],
        out_shape=[jax.ShapeDtypeStruct((n, 16), jnp.float32),
                   jax.ShapeDtypeStruct((n, 16), jnp.float32)],
    )(x, wcat_t)


def _p2(parts, cnts, r, b1l, bm=2000):
    n = r.shape[0]

    def body(p_ref, c_ref, r_ref, b_ref, h_ref, rcp_ref):
        rcp = 1.0 / jnp.maximum(c_ref[0] + c_ref[1], 1.0)
        agg = (p_ref[0] + p_ref[1]) * rcp
        h_ref[...] = jnp.maximum(agg + b_ref[...] + r_ref[...], 0.0)
        rcp_ref[...] = rcp

    return pl.pallas_call(
        body,
        grid=(n // bm,),
        in_specs=[pl.BlockSpec((NC, bm, 16), lambda i: (0, i, 0)),
                  pl.BlockSpec((NC, bm, 16), lambda i: (0, i, 0)),
                  pl.BlockSpec((bm, 16), lambda i: (i, 0)),
                  pl.BlockSpec((1, 16), lambda i: (0, 0))],
        out_specs=[pl.BlockSpec((bm, 16), lambda i: (i, 0)),
                   pl.BlockSpec((bm, 16), lambda i: (i, 0))],
        out_shape=[jax.ShapeDtypeStruct((n, 16), jnp.float32),
                   jax.ShapeDtypeStruct((n, 16), jnp.float32)],
    )(parts, cnts, r, b1l)


def _p3(parts2, rcp, h, w2l_t, b2l, w2r_t, bm=2000):
    n = h.shape[0]
    c_out = w2l_t.shape[1]

    def body(p_ref, rcp_ref, h_ref, wl_ref, b_ref, wr_ref, o_ref):
        agg = (p_ref[0] + p_ref[1]) * rcp_ref[...]
        a = (jnp.dot(agg, wl_ref[...], preferred_element_type=jnp.float32)
             + jnp.dot(h_ref[...], wr_ref[...], preferred_element_type=jnp.float32)
             + b_ref[...])
        m = jnp.max(a, axis=1, keepdims=True)
        lse = m + jnp.log(jnp.sum(jnp.exp(a - m), axis=1, keepdims=True))
        o_ref[...] = a - lse

    return pl.pallas_call(
        body,
        grid=(n // bm,),
        in_specs=[pl.BlockSpec((NC, bm, 16), lambda i: (0, i, 0)),
                  pl.BlockSpec((bm, 16), lambda i: (i, 0)),
                  pl.BlockSpec((bm, 16), lambda i: (i, 0)),
                  pl.BlockSpec((16, c_out), lambda i: (0, 0)),
                  pl.BlockSpec((1, c_out), lambda i: (0, 0)),
                  pl.BlockSpec((16, c_out), lambda i: (0, 0))],
        out_specs=pl.BlockSpec((bm, c_out), lambda i: (i, 0)),
        out_shape=jax.ShapeDtypeStruct((n, c_out), jnp.float32),
    )(parts2, rcp, h, w2l_t, b2l, w2r_t)


def kernel(x, edge_index, edge_weight, W1l, b1l, W1r, W2l, b2l, W2r):
    n, f_in = x.shape
    e = edge_index.shape[1]

    k1, k2 = 4, 8
    quant = NW * CH * k2
    epad = -(-e // quant) * quant
    pad = epad - e
    src2d = jnp.concatenate(
        [edge_index[0], jnp.zeros((pad,), jnp.int32)]).reshape(-1, CH)
    dst2d = jnp.concatenate(
        [edge_index[1], jnp.full((pad,), n, jnp.int32)]).reshape(-1, CH)
    e2d_rows = epad // CH

    nacc = -(-(n + 1) // (NS * 8)) * NS * 8
    zeros_h = jnp.zeros((nacc, 16), jnp.float32)
    ones_h = jnp.ones((CH, 16), jnp.float32)

    wcat_t = jnp.concatenate([W1l, W1r], axis=0).T

    z, r = _p1(x, wcat_t)
    parts, cnts = _make_agg(n, e2d_rows, k1, True)(
        z, src2d, dst2d, zeros_h, ones_h)
    h, rcp = _p2(parts, cnts, r, b1l.reshape(1, -1))
    (parts2,) = _make_agg(n, e2d_rows, k2, False)(
        h, src2d, dst2d, zeros_h)
    return _p3(parts2, rcp, h, W2l.T, b2l.reshape(1, -1), W2r.T)


# speedup vs baseline: 12.6463x; 1.0384x over previous
import functools

import jax
import jax.numpy as jnp
from jax import lax
from jax.experimental import pallas as pl
from jax.experimental.pallas import tpu as pltpu
from jax.experimental.pallas import tpu_sc as plsc

NC = 2
NS = 16
NW = NC * NS
CH = 128


def _make_agg(n_nodes, e2d_rows, k, with_count):
    wrows = e2d_rows // NW
    iters = wrows // k
    assert wrows % k == 0
    nacc = -(-(n_nodes + 1) // (NS * 8)) * NS * 8
    zsl = nacc // NS
    f32 = jnp.float32

    outs = [jax.ShapeDtypeStruct((NC, nacc, 16), f32)]
    scratch = [
        pltpu.VMEM((k, CH), jnp.int32),
        pltpu.VMEM((k, CH), jnp.int32),
        pltpu.VMEM((k * CH, 16), f32),
        pltpu.VMEM_SHARED((nacc, 16), f32),
        [pltpu.SemaphoreType.DMA] * k,
        pltpu.SemaphoreType.DMA,
    ]
    if with_count:
        outs = outs + [jax.ShapeDtypeStruct((NC, nacc, 16), f32)]
        scratch = scratch + [
            pltpu.VMEM((CH, 16), f32),
            pltpu.VMEM_SHARED((nacc, 16), f32),
        ]

    def body(*refs):
        if with_count:
            (table, src2d, dst2d, zeros_h, ones_h, out_acc, out_cnt,
             srcb, dstb, rowsb, acc, sem, ssem, onesb, cacc) = refs
        else:
            (table, src2d, dst2d, zeros_h, out_acc,
             srcb, dstb, rowsb, acc, sem, ssem) = refs
        c = lax.axis_index("c")
        s = lax.axis_index("s")
        w = s * NC + c

        pltpu.sync_copy(zeros_h.at[pl.ds(s * zsl, zsl)], acc.at[pl.ds(s * zsl, zsl)])
        if with_count:
            pltpu.sync_copy(zeros_h.at[pl.ds(s * zsl, zsl)], cacc.at[pl.ds(s * zsl, zsl)])
            pltpu.sync_copy(ones_h, onesb)
        plsc.subcore_barrier()

        base = w * wrows

        def it_body(it, carry):
            r0 = base + it * k
            pltpu.sync_copy(src2d.at[pl.ds(r0, k)], srcb)
            pltpu.sync_copy(dst2d.at[pl.ds(r0, k)], dstb)
            gd = [
                pltpu.async_copy(table.at[srcb.at[j]],
                                 rowsb.at[pl.ds(j * CH, CH)], sem[j])
                for j in range(k)
            ]
            sd = []
            for j in range(k):
                gd[j].wait()
                sd.append(pltpu.async_copy(rowsb.at[pl.ds(j * CH, CH)],
                                           acc.at[dstb.at[j]], ssem, add=True))
                if with_count:
                    sd.append(pltpu.async_copy(onesb, cacc.at[dstb.at[j]],
                                               ssem, add=True))
            for d in sd:
                d.wait()
            return carry

        lax.fori_loop(0, iters, it_body, 0)
        plsc.subcore_barrier()

        pltpu.sync_copy(acc.at[pl.ds(s * zsl, zsl)],
                        out_acc.at[c, pl.ds(s * zsl, zsl)])
        if with_count:
            pltpu.sync_copy(cacc.at[pl.ds(s * zsl, zsl)],
                            out_cnt.at[c, pl.ds(s * zsl, zsl)])

    return pl.kernel(
        body,
        out_type=tuple(outs),
        mesh=plsc.VectorSubcoreMesh(core_axis_name="c", subcore_axis_name="s"),
        scratch_types=tuple(scratch),
        compiler_params=pltpu.CompilerParams(use_tc_tiling_on_sc=False),
    )


def _p1(x, wcat_t, bm=2000):
    n, f_in = x.shape

    def body(x_ref, w_ref, z_ref, r_ref):
        res = jnp.dot(x_ref[...], w_ref[...], preferred_element_type=jnp.float32)
        z_ref[...] = res[:, :16]
        r_ref[...] = res[:, 16:]

    return pl.pallas_call(
        body,
        grid=(n // bm,),
        in_specs=[pl.BlockSpec((bm, f_in), lambda i: (i, 0)),
                  pl.BlockSpec((f_in, 32), lambda i: (0, 0))],
        out_specs=[pl.BlockSpec((bm, 16), lambda i: (i, 0)),
                   pl.BlockSpec((bm, 16), lambda i: (i, 0))],
        out_shape=[jax.ShapeDtypeStruct((n, 16), jnp.float32),
                   jax.ShapeDtypeStruct((n, 16), jnp.float32)],
    )(x, wcat_t)


def _p2(parts, cnts, r, b1l, bm=2000):
    n = r.shape[0]

    def body(p_ref, c_ref, r_ref, b_ref, h_ref, rcp_ref):
        rcp = 1.0 / jnp.maximum(c_ref[0] + c_ref[1], 1.0)
        agg = (p_ref[0] + p_ref[1]) * rcp
        h_ref[...] = jnp.maximum(agg + b_ref[...] + r_ref[...], 0.0)
        rcp_ref[...] = rcp

    return pl.pallas_call(
        body,
        grid=(n // bm,),
        in_specs=[pl.BlockSpec((NC, bm, 16), lambda i: (0, i, 0)),
                  pl.BlockSpec((NC, bm, 16), lambda i: (0, i, 0)),
                  pl.BlockSpec((bm, 16), lambda i: (i, 0)),
                  pl.BlockSpec((1, 16), lambda i: (0, 0))],
        out_specs=[pl.BlockSpec((bm, 16), lambda i: (i, 0)),
                   pl.BlockSpec((bm, 16), lambda i: (i, 0))],
        out_shape=[jax.ShapeDtypeStruct((n, 16), jnp.float32),
                   jax.ShapeDtypeStruct((n, 16), jnp.float32)],
    )(parts, cnts, r, b1l)


def _p3(parts2, rcp, h, w2l_t, b2l, w2r_t, bm=2000):
    n = h.shape[0]
    c_out = w2l_t.shape[1]

    def body(p_ref, rcp_ref, h_ref, wl_ref, b_ref, wr_ref, o_ref):
        agg = (p_ref[0] + p_ref[1]) * rcp_ref[...]
        a = (jnp.dot(agg, wl_ref[...], preferred_element_type=jnp.float32)
             + jnp.dot(h_ref[...], wr_ref[...], preferred_element_type=jnp.float32)
             + b_ref[...])
        m = jnp.max(a, axis=1, keepdims=True)
        lse = m + jnp.log(jnp.sum(jnp.exp(a - m), axis=1, keepdims=True))
        o_ref[...] = a - lse

    return pl.pallas_call(
        body,
        grid=(n // bm,),
        in_specs=[pl.BlockSpec((NC, bm, 16), lambda i: (0, i, 0)),
                  pl.BlockSpec((bm, 16), lambda i: (i, 0)),
                  pl.BlockSpec((bm, 16), lambda i: (i, 0)),
                  pl.BlockSpec((16, c_out), lambda i: (0, 0)),
                  pl.BlockSpec((1, c_out), lambda i: (0, 0)),
                  pl.BlockSpec((16, c_out), lambda i: (0, 0))],
        out_specs=pl.BlockSpec((bm, c_out), lambda i: (i, 0)),
        out_shape=jax.ShapeDtypeStruct((n, c_out), jnp.float32),
    )(parts2, rcp, h, w2l_t, b2l, w2r_t)


def kernel(x, edge_index, edge_weight, W1l, b1l, W1r, W2l, b2l, W2r):
    n, f_in = x.shape
    e = edge_index.shape[1]

    k1, k2 = 4, 8
    quant = NW * CH * k2
    epad = -(-e // quant) * quant
    pad = epad - e
    src2d = jnp.concatenate(
        [edge_index[0], jnp.zeros((pad,), jnp.int32)]).reshape(-1, CH)
    dst2d = jnp.concatenate(
        [edge_index[1], jnp.full((pad,), n, jnp.int32)]).reshape(-1, CH)
    e2d_rows = epad // CH

    nacc = -(-(n + 1) // (NS * 8)) * NS * 8
    zeros_h = jnp.zeros((nacc, 16), jnp.float32)
    ones_h = jnp.ones((CH, 16), jnp.float32)

    wcat_t = jnp.concatenate([W1l, W1r], axis=0).T

    z, r = _p1(x, wcat_t)
    parts, cnts = _make_agg(n, e2d_rows, k1, True)(
        z, src2d, dst2d, zeros_h, ones_h)
    h, rcp = _p2(parts, cnts, r, b1l.reshape(1, -1))
    (parts2,) = _make_agg(n, e2d_rows, k2, False)(
        h, src2d, dst2d, zeros_h)
    return _p3(parts2, rcp, h, W2l.T, b2l.reshape(1, -1), W2r.T)


# speedup vs baseline: 14.4772x; 1.1448x over previous
import functools

import jax
import jax.numpy as jnp
from jax import lax
from jax.experimental import pallas as pl
from jax.experimental.pallas import tpu as pltpu
from jax.experimental.pallas import tpu_sc as plsc

NC = 2
NS = 16
NW = NC * NS
CH = 128


def _make_agg(n_nodes, e2d_rows, k, with_count):
    wrows = e2d_rows // NW
    n_groups = wrows // k
    assert wrows % k == 0 and n_groups % 2 == 0
    nacc = -(-(n_nodes + 1) // (NS * 8)) * NS * 8
    zsl = nacc // NS
    f32 = jnp.float32

    outs = [jax.ShapeDtypeStruct((NC, nacc, 16), f32)]
    scratch = [
        pltpu.VMEM((k, CH), jnp.int32),
        pltpu.VMEM((k, CH), jnp.int32),
        pltpu.VMEM((k * CH, 16), f32),
        pltpu.VMEM((k, CH), jnp.int32),
        pltpu.VMEM((k, CH), jnp.int32),
        pltpu.VMEM((k * CH, 16), f32),
        pltpu.VMEM_SHARED((nacc, 16), f32),
        pltpu.SemaphoreType.DMA,
        pltpu.SemaphoreType.DMA,
        pltpu.SemaphoreType.DMA,
        pltpu.SemaphoreType.DMA,
    ]
    if with_count:
        outs = outs + [jax.ShapeDtypeStruct((NC, nacc, 16), f32)]
        scratch = scratch + [
            pltpu.VMEM((CH, 16), f32),
            pltpu.VMEM_SHARED((nacc, 16), f32),
        ]

    def body(*refs):
        if with_count:
            (table, src2d, dst2d, zeros_h, ones_h, out_acc, out_cnt,
             srcbA, dstbA, rowsbA, srcbB, dstbB, rowsbB, acc,
             gsemA, gsemB, ssemA, ssemB, onesb, cacc) = refs
        else:
            (table, src2d, dst2d, zeros_h, out_acc,
             srcbA, dstbA, rowsbA, srcbB, dstbB, rowsbB, acc,
             gsemA, gsemB, ssemA, ssemB) = refs
        c = lax.axis_index("c")
        s = lax.axis_index("s")
        w = s * NC + c

        pltpu.sync_copy(zeros_h.at[pl.ds(s * zsl, zsl)], acc.at[pl.ds(s * zsl, zsl)])
        if with_count:
            pltpu.sync_copy(zeros_h.at[pl.ds(s * zsl, zsl)], cacc.at[pl.ds(s * zsl, zsl)])
            pltpu.sync_copy(ones_h, onesb)
        plsc.subcore_barrier()

        base = w * wrows

        def load_group(g, srcb, dstb, rowsb, gsem):
            r0 = base + g * k
            pltpu.sync_copy(src2d.at[pl.ds(r0, k)], srcb)
            pltpu.sync_copy(dst2d.at[pl.ds(r0, k)], dstb)
            for j in range(k):
                pltpu.async_copy(table.at[srcb.at[j]],
                                 rowsb.at[pl.ds(j * CH, CH)], gsem)

        def drain_gathers(srcb, rowsb, gsem):
            for j in range(k):
                pltpu.make_async_copy(table.at[srcb.at[j]],
                                      rowsb.at[pl.ds(j * CH, CH)], gsem).wait()

        def fire_scatters(dstb, rowsb, ssem):
            sd = [pltpu.async_copy(rowsb.at[pl.ds(j * CH, CH)],
                                   acc.at[dstb.at[j]], ssem, add=True)
                  for j in range(k)]
            if with_count:
                sd += [pltpu.async_copy(onesb, cacc.at[dstb.at[j]], ssem,
                                        add=True)
                       for j in range(k)]
            return sd

        load_group(0, srcbA, dstbA, rowsbA, gsemA)
        load_group(1, srcbB, dstbB, rowsbB, gsemB)

        def it_body(t, carry):
            g0 = 2 * t
            drain_gathers(srcbA, rowsbA, gsemA)
            sdA = fire_scatters(dstbA, rowsbA, ssemA)
            drain_gathers(srcbB, rowsbB, gsemB)
            sdB = fire_scatters(dstbB, rowsbB, ssemB)
            for d in sdA:
                d.wait()

            @pl.when(g0 + 2 < n_groups)
            def _():
                load_group(g0 + 2, srcbA, dstbA, rowsbA, gsemA)

            for d in sdB:
                d.wait()

            @pl.when(g0 + 3 < n_groups)
            def _():
                load_group(g0 + 3, srcbB, dstbB, rowsbB, gsemB)

            return carry

        lax.fori_loop(0, n_groups // 2, it_body, 0)
        plsc.subcore_barrier()

        pltpu.sync_copy(acc.at[pl.ds(s * zsl, zsl)],
                        out_acc.at[c, pl.ds(s * zsl, zsl)])
        if with_count:
            pltpu.sync_copy(cacc.at[pl.ds(s * zsl, zsl)],
                            out_cnt.at[c, pl.ds(s * zsl, zsl)])

    return pl.kernel(
        body,
        out_type=tuple(outs),
        mesh=plsc.VectorSubcoreMesh(core_axis_name="c", subcore_axis_name="s"),
        scratch_types=tuple(scratch),
        compiler_params=pltpu.CompilerParams(use_tc_tiling_on_sc=False),
    )


def _p1(x, wcat_t, bm=2000):
    n, f_in = x.shape

    def body(x_ref, w_ref, z_ref, r_ref):
        res = jnp.dot(x_ref[...], w_ref[...], preferred_element_type=jnp.float32)
        z_ref[...] = res[:, :16]
        r_ref[...] = res[:, 16:]

    return pl.pallas_call(
        body,
        grid=(n // bm,),
        in_specs=[pl.BlockSpec((bm, f_in), lambda i: (i, 0)),
                  pl.BlockSpec((f_in, 32), lambda i: (0, 0))],
        out_specs=[pl.BlockSpec((bm, 16), lambda i: (i, 0)),
                   pl.BlockSpec((bm, 16), lambda i: (i, 0))],
        out_shape=[jax.ShapeDtypeStruct((n, 16), jnp.float32),
                   jax.ShapeDtypeStruct((n, 16), jnp.float32)],
    )(x, wcat_t)


def _p2(parts, cnts, r, b1l, bm=2000):
    n = r.shape[0]

    def body(p_ref, c_ref, r_ref, b_ref, h_ref, rcp_ref):
        rcp = 1.0 / jnp.maximum(c_ref[0] + c_ref[1], 1.0)
        agg = (p_ref[0] + p_ref[1]) * rcp
        h_ref[...] = jnp.maximum(agg + b_ref[...] + r_ref[...], 0.0)
        rcp_ref[...] = rcp

    return pl.pallas_call(
        body,
        grid=(n // bm,),
        in_specs=[pl.BlockSpec((NC, bm, 16), lambda i: (0, i, 0)),
                  pl.BlockSpec((NC, bm, 16), lambda i: (0, i, 0)),
                  pl.BlockSpec((bm, 16), lambda i: (i, 0)),
                  pl.BlockSpec((1, 16), lambda i: (0, 0))],
        out_specs=[pl.BlockSpec((bm, 16), lambda i: (i, 0)),
                   pl.BlockSpec((bm, 16), lambda i: (i, 0))],
        out_shape=[jax.ShapeDtypeStruct((n, 16), jnp.float32),
                   jax.ShapeDtypeStruct((n, 16), jnp.float32)],
    )(parts, cnts, r, b1l)


def _p3(parts2, rcp, h, w2l_t, b2l, w2r_t, bm=2000):
    n = h.shape[0]
    c_out = w2l_t.shape[1]

    def body(p_ref, rcp_ref, h_ref, wl_ref, b_ref, wr_ref, o_ref):
        agg = (p_ref[0] + p_ref[1]) * rcp_ref[...]
        a = (jnp.dot(agg, wl_ref[...], preferred_element_type=jnp.float32)
             + jnp.dot(h_ref[...], wr_ref[...], preferred_element_type=jnp.float32)
             + b_ref[...])
        m = jnp.max(a, axis=1, keepdims=True)
        lse = m + jnp.log(jnp.sum(jnp.exp(a - m), axis=1, keepdims=True))
        o_ref[...] = a - lse

    return pl.pallas_call(
        body,
        grid=(n // bm,),
        in_specs=[pl.BlockSpec((NC, bm, 16), lambda i: (0, i, 0)),
                  pl.BlockSpec((bm, 16), lambda i: (i, 0)),
                  pl.BlockSpec((bm, 16), lambda i: (i, 0)),
                  pl.BlockSpec((16, c_out), lambda i: (0, 0)),
                  pl.BlockSpec((1, c_out), lambda i: (0, 0)),
                  pl.BlockSpec((16, c_out), lambda i: (0, 0))],
        out_specs=pl.BlockSpec((bm, c_out), lambda i: (i, 0)),
        out_shape=jax.ShapeDtypeStruct((n, c_out), jnp.float32),
    )(parts2, rcp, h, w2l_t, b2l, w2r_t)


def kernel(x, edge_index, edge_weight, W1l, b1l, W1r, W2l, b2l, W2r):
    n, f_in = x.shape
    e = edge_index.shape[1]

    k1, k2 = 4, 8
    quant = NW * CH * k2
    epad = -(-e // quant) * quant
    pad = epad - e
    src2d = jnp.concatenate(
        [edge_index[0], jnp.zeros((pad,), jnp.int32)]).reshape(-1, CH)
    dst2d = jnp.concatenate(
        [edge_index[1], jnp.full((pad,), n, jnp.int32)]).reshape(-1, CH)
    e2d_rows = epad // CH

    nacc = -(-(n + 1) // (NS * 8)) * NS * 8
    zeros_h = jnp.zeros((nacc, 16), jnp.float32)
    ones_h = jnp.ones((CH, 16), jnp.float32)

    wcat_t = jnp.concatenate([W1l, W1r], axis=0).T

    z, r = _p1(x, wcat_t)
    parts, cnts = _make_agg(n, e2d_rows, k1, True)(
        z, src2d, dst2d, zeros_h, ones_h)
    h, rcp = _p2(parts, cnts, r, b1l.reshape(1, -1))
    (parts2,) = _make_agg(n, e2d_rows, k2, False)(
        h, src2d, dst2d, zeros_h)
    return _p3(parts2, rcp, h, W2l.T, b2l.reshape(1, -1), W2r.T)


# speedup vs baseline: 15.3442x; 1.0599x over previous
import functools

import jax
import jax.numpy as jnp
from jax import lax
from jax.experimental import pallas as pl
from jax.experimental.pallas import tpu as pltpu
from jax.experimental.pallas import tpu_sc as plsc

NC = 2
NS = 16
NW = NC * NS
CH = 128


def _make_agg(n_nodes, e2d_rows, k, with_count, split=0.5):
    pair_groups = e2d_rows // (NS * k)
    g0 = 2 * round(split * pair_groups / 2)
    g1 = pair_groups - g0
    assert e2d_rows % (NS * k) == 0 and g1 % 2 == 0 and g0 >= 2 and g1 >= 2
    rows0, rows1 = g0 * k, g1 * k
    nacc = -(-(n_nodes + 1) // (NS * 8)) * NS * 8
    zsl = nacc // NS
    f32 = jnp.float32

    outs = [jax.ShapeDtypeStruct((NC, nacc, 16), f32)]
    scratch = [
        pltpu.VMEM((k, CH), jnp.int32),
        pltpu.VMEM((k, CH), jnp.int32),
        pltpu.VMEM((k * CH, 16), f32),
        pltpu.VMEM((k, CH), jnp.int32),
        pltpu.VMEM((k, CH), jnp.int32),
        pltpu.VMEM((k * CH, 16), f32),
        pltpu.VMEM_SHARED((nacc, 16), f32),
        pltpu.SemaphoreType.DMA,
        pltpu.SemaphoreType.DMA,
        pltpu.SemaphoreType.DMA,
        pltpu.SemaphoreType.DMA,
    ]
    if with_count:
        outs = outs + [jax.ShapeDtypeStruct((NC, nacc, 16), f32)]
        scratch = scratch + [
            pltpu.VMEM((CH, 16), f32),
            pltpu.VMEM_SHARED((nacc, 16), f32),
        ]

    def body(*refs):
        if with_count:
            (table, src2d, dst2d, zeros_h, ones_h, out_acc, out_cnt,
             srcbA, dstbA, rowsbA, srcbB, dstbB, rowsbB, acc,
             gsemA, gsemB, ssemA, ssemB, onesb, cacc) = refs
        else:
            (table, src2d, dst2d, zeros_h, out_acc,
             srcbA, dstbA, rowsbA, srcbB, dstbB, rowsbB, acc,
             gsemA, gsemB, ssemA, ssemB) = refs
        c = lax.axis_index("c")
        s = lax.axis_index("s")
        n_groups = jnp.where(c == 0, g0, g1)

        pltpu.sync_copy(zeros_h.at[pl.ds(s * zsl, zsl)], acc.at[pl.ds(s * zsl, zsl)])
        if with_count:
            pltpu.sync_copy(zeros_h.at[pl.ds(s * zsl, zsl)], cacc.at[pl.ds(s * zsl, zsl)])
            pltpu.sync_copy(ones_h, onesb)
        plsc.subcore_barrier()

        base = jnp.where(c == 0, s * rows0, NS * rows0 + s * rows1)

        def load_group(g, srcb, dstb, rowsb, gsem):
            r0 = base + g * k
            pltpu.sync_copy(src2d.at[pl.ds(r0, k)], srcb)
            pltpu.sync_copy(dst2d.at[pl.ds(r0, k)], dstb)
            for j in range(k):
                pltpu.async_copy(table.at[srcb.at[j]],
                                 rowsb.at[pl.ds(j * CH, CH)], gsem)

        def drain_gathers(srcb, rowsb, gsem):
            for j in range(k):
                pltpu.make_async_copy(table.at[srcb.at[j]],
                                      rowsb.at[pl.ds(j * CH, CH)], gsem).wait()

        def fire_scatters(dstb, rowsb, ssem):
            sd = [pltpu.async_copy(rowsb.at[pl.ds(j * CH, CH)],
                                   acc.at[dstb.at[j]], ssem, add=True)
                  for j in range(k)]
            if with_count:
                sd += [pltpu.async_copy(onesb, cacc.at[dstb.at[j]], ssem,
                                        add=True)
                       for j in range(k)]
            return sd

        load_group(0, srcbA, dstbA, rowsbA, gsemA)
        load_group(1, srcbB, dstbB, rowsbB, gsemB)

        def it_body(t, carry):
            ga = 2 * t
            drain_gathers(srcbA, rowsbA, gsemA)
            sdA = fire_scatters(dstbA, rowsbA, ssemA)
            drain_gathers(srcbB, rowsbB, gsemB)
            sdB = fire_scatters(dstbB, rowsbB, ssemB)
            for d in sdA:
                d.wait()

            @pl.when(ga + 2 < n_groups)
            def _():
                load_group(ga + 2, srcbA, dstbA, rowsbA, gsemA)

            for d in sdB:
                d.wait()

            @pl.when(ga + 3 < n_groups)
            def _():
                load_group(ga + 3, srcbB, dstbB, rowsbB, gsemB)

            return carry

        lax.fori_loop(0, n_groups // 2, it_body, 0)
        plsc.subcore_barrier()

        pltpu.sync_copy(acc.at[pl.ds(s * zsl, zsl)],
                        out_acc.at[c, pl.ds(s * zsl, zsl)])
        if with_count:
            pltpu.sync_copy(cacc.at[pl.ds(s * zsl, zsl)],
                            out_cnt.at[c, pl.ds(s * zsl, zsl)])

    return pl.kernel(
        body,
        out_type=tuple(outs),
        mesh=plsc.VectorSubcoreMesh(core_axis_name="c", subcore_axis_name="s"),
        scratch_types=tuple(scratch),
        compiler_params=pltpu.CompilerParams(use_tc_tiling_on_sc=False),
    )


def _p1(x, wcat_t, bm=2000):
    n, f_in = x.shape

    def body(x_ref, w_ref, z_ref, r_ref):
        res = jnp.dot(x_ref[...], w_ref[...], preferred_element_type=jnp.float32)
        z_ref[...] = res[:, :16]
        r_ref[...] = res[:, 16:]

    return pl.pallas_call(
        body,
        grid=(n // bm,),
        in_specs=[pl.BlockSpec((bm, f_in), lambda i: (i, 0)),
                  pl.BlockSpec((f_in, 32), lambda i: (0, 0))],
        out_specs=[pl.BlockSpec((bm, 16), lambda i: (i, 0)),
                   pl.BlockSpec((bm, 16), lambda i: (i, 0))],
        out_shape=[jax.ShapeDtypeStruct((n, 16), jnp.float32),
                   jax.ShapeDtypeStruct((n, 16), jnp.float32)],
    )(x, wcat_t)


def _p2(parts, cnts, r, b1l, bm=2000):
    n = r.shape[0]

    def body(p_ref, c_ref, r_ref, b_ref, h_ref, rcp_ref):
        rcp = 1.0 / jnp.maximum(c_ref[0] + c_ref[1], 1.0)
        agg = (p_ref[0] + p_ref[1]) * rcp
        h_ref[...] = jnp.maximum(agg + b_ref[...] + r_ref[...], 0.0)
        rcp_ref[...] = rcp

    return pl.pallas_call(
        body,
        grid=(n // bm,),
        in_specs=[pl.BlockSpec((NC, bm, 16), lambda i: (0, i, 0)),
                  pl.BlockSpec((NC, bm, 16), lambda i: (0, i, 0)),
                  pl.BlockSpec((bm, 16), lambda i: (i, 0)),
                  pl.BlockSpec((1, 16), lambda i: (0, 0))],
        out_specs=[pl.BlockSpec((bm, 16), lambda i: (i, 0)),
                   pl.BlockSpec((bm, 16), lambda i: (i, 0))],
        out_shape=[jax.ShapeDtypeStruct((n, 16), jnp.float32),
                   jax.ShapeDtypeStruct((n, 16), jnp.float32)],
    )(parts, cnts, r, b1l)


def _p3(parts2, rcp, h, w2l_t, b2l, w2r_t, bm=2000):
    n = h.shape[0]
    c_out = w2l_t.shape[1]

    def body(p_ref, rcp_ref, h_ref, wl_ref, b_ref, wr_ref, o_ref):
        agg = (p_ref[0] + p_ref[1]) * rcp_ref[...]
        a = (jnp.dot(agg, wl_ref[...], preferred_element_type=jnp.float32)
             + jnp.dot(h_ref[...], wr_ref[...], preferred_element_type=jnp.float32)
             + b_ref[...])
        m = jnp.max(a, axis=1, keepdims=True)
        lse = m + jnp.log(jnp.sum(jnp.exp(a - m), axis=1, keepdims=True))
        o_ref[...] = a - lse

    return pl.pallas_call(
        body,
        grid=(n // bm,),
        in_specs=[pl.BlockSpec((NC, bm, 16), lambda i: (0, i, 0)),
                  pl.BlockSpec((bm, 16), lambda i: (i, 0)),
                  pl.BlockSpec((bm, 16), lambda i: (i, 0)),
                  pl.BlockSpec((16, c_out), lambda i: (0, 0)),
                  pl.BlockSpec((1, c_out), lambda i: (0, 0)),
                  pl.BlockSpec((16, c_out), lambda i: (0, 0))],
        out_specs=pl.BlockSpec((bm, c_out), lambda i: (i, 0)),
        out_shape=jax.ShapeDtypeStruct((n, c_out), jnp.float32),
    )(parts2, rcp, h, w2l_t, b2l, w2r_t)


def kernel(x, edge_index, edge_weight, W1l, b1l, W1r, W2l, b2l, W2r):
    n, f_in = x.shape
    e = edge_index.shape[1]

    k1, k2 = 4, 8
    quant = NW * CH * k2
    epad = -(-e // quant) * quant
    pad = epad - e
    src2d = jnp.concatenate(
        [edge_index[0], jnp.zeros((pad,), jnp.int32)]).reshape(-1, CH)
    dst2d = jnp.concatenate(
        [edge_index[1], jnp.full((pad,), n, jnp.int32)]).reshape(-1, CH)
    e2d_rows = epad // CH

    nacc = -(-(n + 1) // (NS * 8)) * NS * 8
    zeros_h = jnp.zeros((nacc, 16), jnp.float32)
    ones_h = jnp.ones((CH, 16), jnp.float32)

    wcat_t = jnp.concatenate([W1l, W1r], axis=0).T

    z, r = _p1(x, wcat_t)
    parts, cnts = _make_agg(n, e2d_rows, k1, True, split=0.67)(
        z, src2d, dst2d, zeros_h, ones_h)
    h, rcp = _p2(parts, cnts, r, b1l.reshape(1, -1))
    (parts2,) = _make_agg(n, e2d_rows, k2, False, split=0.67)(
        h, src2d, dst2d, zeros_h)
    return _p3(parts2, rcp, h, W2l.T, b2l.reshape(1, -1), W2r.T)


# speedup vs baseline: 16.0004x; 1.0428x over previous
import functools

import jax
import jax.numpy as jnp
from jax import lax
from jax.experimental import pallas as pl
from jax.experimental.pallas import tpu as pltpu
from jax.experimental.pallas import tpu_sc as plsc

NC = 2
NS = 16
NW = NC * NS
CH = 128


def _make_agg(n_nodes, e2d_rows, k, with_count, split=0.5):
    pair_groups = e2d_rows // (NS * k)
    g0 = 2 * round(split * pair_groups / 2)
    g1 = pair_groups - g0
    assert e2d_rows % (NS * k) == 0 and g1 % 2 == 0 and g0 >= 2 and g1 >= 2
    rows0, rows1 = g0 * k, g1 * k
    nacc = -(-(n_nodes + 1) // (NS * 8)) * NS * 8
    zsl = nacc // NS
    f32 = jnp.float32

    outs = [jax.ShapeDtypeStruct((NC, nacc, 16), f32)]
    scratch = [
        pltpu.VMEM((k, CH), jnp.int32),
        pltpu.VMEM((k, CH), jnp.int32),
        pltpu.VMEM((k * CH, 16), f32),
        pltpu.VMEM((k, CH), jnp.int32),
        pltpu.VMEM((k, CH), jnp.int32),
        pltpu.VMEM((k * CH, 16), f32),
        pltpu.VMEM_SHARED((nacc, 16), f32),
        pltpu.SemaphoreType.DMA,
        pltpu.SemaphoreType.DMA,
        pltpu.SemaphoreType.DMA,
        pltpu.SemaphoreType.DMA,
    ]
    if with_count:
        outs = outs + [jax.ShapeDtypeStruct((NC, nacc, 16), f32)]
        scratch = scratch + [
            pltpu.VMEM((CH, 16), f32),
            pltpu.VMEM_SHARED((nacc, 16), f32),
        ]

    def body(*refs):
        if with_count:
            (table, src2d, dst2d, zeros_h, ones_h, out_acc, out_cnt,
             srcbA, dstbA, rowsbA, srcbB, dstbB, rowsbB, acc,
             gsemA, gsemB, ssemA, ssemB, onesb, cacc) = refs
        else:
            (table, src2d, dst2d, zeros_h, out_acc,
             srcbA, dstbA, rowsbA, srcbB, dstbB, rowsbB, acc,
             gsemA, gsemB, ssemA, ssemB) = refs
        c = lax.axis_index("c")
        s = lax.axis_index("s")
        n_groups = jnp.where(c == 0, g0, g1)

        pltpu.sync_copy(zeros_h.at[pl.ds(s * zsl, zsl)], acc.at[pl.ds(s * zsl, zsl)])
        if with_count:
            pltpu.sync_copy(zeros_h.at[pl.ds(s * zsl, zsl)], cacc.at[pl.ds(s * zsl, zsl)])
            pltpu.sync_copy(ones_h, onesb)
        plsc.subcore_barrier()

        base = jnp.where(c == 0, s * rows0, NS * rows0 + s * rows1)

        def load_group(g, srcb, dstb, rowsb, gsem):
            r0 = base + g * k
            pltpu.sync_copy(src2d.at[pl.ds(r0, k)], srcb)
            pltpu.sync_copy(dst2d.at[pl.ds(r0, k)], dstb)
            for j in range(k):
                pltpu.async_copy(table.at[srcb.at[j]],
                                 rowsb.at[pl.ds(j * CH, CH)], gsem)

        def drain_gathers(srcb, rowsb, gsem):
            for j in range(k):
                pltpu.make_async_copy(table.at[srcb.at[j]],
                                      rowsb.at[pl.ds(j * CH, CH)], gsem).wait()

        def fire_scatters(dstb, rowsb, ssem):
            sd = [pltpu.async_copy(rowsb.at[pl.ds(j * CH, CH)],
                                   acc.at[dstb.at[j]], ssem, add=True)
                  for j in range(k)]
            if with_count:
                sd += [pltpu.async_copy(onesb, cacc.at[dstb.at[j]], ssem,
                                        add=True)
                       for j in range(k)]
            return sd

        load_group(0, srcbA, dstbA, rowsbA, gsemA)
        load_group(1, srcbB, dstbB, rowsbB, gsemB)

        def it_body(t, carry):
            ga = 2 * t
            drain_gathers(srcbA, rowsbA, gsemA)
            sdA = fire_scatters(dstbA, rowsbA, ssemA)
            drain_gathers(srcbB, rowsbB, gsemB)
            sdB = fire_scatters(dstbB, rowsbB, ssemB)
            for d in sdA:
                d.wait()

            @pl.when(ga + 2 < n_groups)
            def _():
                load_group(ga + 2, srcbA, dstbA, rowsbA, gsemA)

            for d in sdB:
                d.wait()

            @pl.when(ga + 3 < n_groups)
            def _():
                load_group(ga + 3, srcbB, dstbB, rowsbB, gsemB)

            return carry

        lax.fori_loop(0, n_groups // 2, it_body, 0)
        plsc.subcore_barrier()

        pltpu.sync_copy(acc.at[pl.ds(s * zsl, zsl)],
                        out_acc.at[c, pl.ds(s * zsl, zsl)])
        if with_count:
            pltpu.sync_copy(cacc.at[pl.ds(s * zsl, zsl)],
                            out_cnt.at[c, pl.ds(s * zsl, zsl)])

    return pl.kernel(
        body,
        out_type=tuple(outs),
        mesh=plsc.VectorSubcoreMesh(core_axis_name="c", subcore_axis_name="s"),
        scratch_types=tuple(scratch),
        compiler_params=pltpu.CompilerParams(use_tc_tiling_on_sc=False),
    )


def _p1(x, wcat_t, bm=2000):
    n, f_in = x.shape

    def body(x_ref, w_ref, z_ref, r_ref):
        res = jnp.dot(x_ref[...], w_ref[...], preferred_element_type=jnp.float32)
        z_ref[...] = res[:, :16]
        r_ref[...] = res[:, 16:]

    return pl.pallas_call(
        body,
        grid=(n // bm,),
        in_specs=[pl.BlockSpec((bm, f_in), lambda i: (i, 0)),
                  pl.BlockSpec((f_in, 32), lambda i: (0, 0))],
        out_specs=[pl.BlockSpec((bm, 16), lambda i: (i, 0)),
                   pl.BlockSpec((bm, 16), lambda i: (i, 0))],
        out_shape=[jax.ShapeDtypeStruct((n, 16), jnp.float32),
                   jax.ShapeDtypeStruct((n, 16), jnp.float32)],
    )(x, wcat_t)


def _p2(parts, cnts, r, b1l, bm=2000):
    n = r.shape[0]

    def body(p_ref, c_ref, r_ref, b_ref, h_ref, rcp_ref):
        rcp = 1.0 / jnp.maximum(c_ref[0] + c_ref[1], 1.0)
        agg = (p_ref[0] + p_ref[1]) * rcp
        h_ref[...] = jnp.maximum(agg + b_ref[...] + r_ref[...], 0.0)
        rcp_ref[...] = rcp

    return pl.pallas_call(
        body,
        grid=(n // bm,),
        in_specs=[pl.BlockSpec((NC, bm, 16), lambda i: (0, i, 0)),
                  pl.BlockSpec((NC, bm, 16), lambda i: (0, i, 0)),
                  pl.BlockSpec((bm, 16), lambda i: (i, 0)),
                  pl.BlockSpec((1, 16), lambda i: (0, 0))],
        out_specs=[pl.BlockSpec((bm, 16), lambda i: (i, 0)),
                   pl.BlockSpec((bm, 16), lambda i: (i, 0))],
        out_shape=[jax.ShapeDtypeStruct((n, 16), jnp.float32),
                   jax.ShapeDtypeStruct((n, 16), jnp.float32)],
    )(parts, cnts, r, b1l)


def _p3(parts2, rcp, h, w2l_t, b2l, w2r_t, bm=2000):
    n = h.shape[0]
    c_out = w2l_t.shape[1]

    def body(p_ref, rcp_ref, h_ref, wl_ref, b_ref, wr_ref, o_ref):
        agg = (p_ref[0] + p_ref[1]) * rcp_ref[...]
        a = (jnp.dot(agg, wl_ref[...], preferred_element_type=jnp.float32)
             + jnp.dot(h_ref[...], wr_ref[...], preferred_element_type=jnp.float32)
             + b_ref[...])
        m = jnp.max(a, axis=1, keepdims=True)
        lse = m + jnp.log(jnp.sum(jnp.exp(a - m), axis=1, keepdims=True))
        o_ref[...] = a - lse

    return pl.pallas_call(
        body,
        grid=(n // bm,),
        in_specs=[pl.BlockSpec((NC, bm, 16), lambda i: (0, i, 0)),
                  pl.BlockSpec((bm, 16), lambda i: (i, 0)),
                  pl.BlockSpec((bm, 16), lambda i: (i, 0)),
                  pl.BlockSpec((16, c_out), lambda i: (0, 0)),
                  pl.BlockSpec((1, c_out), lambda i: (0, 0)),
                  pl.BlockSpec((16, c_out), lambda i: (0, 0))],
        out_specs=pl.BlockSpec((bm, c_out), lambda i: (i, 0)),
        out_shape=jax.ShapeDtypeStruct((n, c_out), jnp.float32),
    )(parts2, rcp, h, w2l_t, b2l, w2r_t)


def kernel(x, edge_index, edge_weight, W1l, b1l, W1r, W2l, b2l, W2r):
    n, f_in = x.shape
    e = edge_index.shape[1]

    k1, k2 = 4, 8
    quant = NW * CH * k2
    epad = -(-e // quant) * quant
    pad = epad - e
    src2d = jnp.concatenate(
        [edge_index[0], jnp.zeros((pad,), jnp.int32)]).reshape(-1, CH)
    dst2d = jnp.concatenate(
        [edge_index[1], jnp.full((pad,), n, jnp.int32)]).reshape(-1, CH)
    e2d_rows = epad // CH

    nacc = -(-(n + 1) // (NS * 8)) * NS * 8
    zeros_h = jnp.zeros((nacc, 16), jnp.float32)
    ones_h = jnp.ones((CH, 16), jnp.float32)

    wcat_t = jnp.concatenate([W1l, W1r], axis=0).T

    z, r = _p1(x, wcat_t)
    parts, cnts = _make_agg(n, e2d_rows, k1, True, split=0.8)(
        z, src2d, dst2d, zeros_h, ones_h)
    h, rcp = _p2(parts, cnts, r, b1l.reshape(1, -1))
    (parts2,) = _make_agg(n, e2d_rows, k2, False, split=0.8)(
        h, src2d, dst2d, zeros_h)
    return _p3(parts2, rcp, h, W2l.T, b2l.reshape(1, -1), W2r.T)
